# Initial kernel scaffold; baseline (speedup 1.0000x reference)
import functools

import jax
import jax.numpy as jnp
from jax import lax
from jax.experimental import pallas as pl
from jax.experimental.pallas import tpu as pltpu
from jax.experimental.pallas import tpu_sc as plsc

_N_COARSE = 10000
_N_FINE = 50000
_VAL_DIM = 128
_FE = 9
_NF = 64

_NC = 2
_NS = 16
_NW = _NC * _NS
_BPW = 1568
_NPAD = _NW * _BPW
_C = 56
_NCHUNK = _BPW // _C


def _mm_body(x_ref, w_ref, o_ref):
    o_ref[...] = jnp.dot(x_ref[...], w_ref[...],
                         preferred_element_type=jnp.float32)


def _project_table(table, w2):
    return pl.pallas_call(
        _mm_body,
        grid=(10,),
        in_specs=[
            pl.BlockSpec((1000, _VAL_DIM), lambda i: (i, 0)),
            pl.BlockSpec((_VAL_DIM, _FE * _NF), lambda i: (0, 0)),
        ],
        out_specs=pl.BlockSpec((1000, _FE * _NF), lambda i: (i, 0)),
        out_shape=jax.ShapeDtypeStruct((_N_COARSE, _FE * _NF), jnp.float32),
    )(table, w2)


def _sc_gather_sum(p_flat, idx_l):
    mesh = plsc.VectorSubcoreMesh(core_axis_name="c", subcore_axis_name="s")

    @functools.partial(
        pl.kernel,
        mesh=mesh,
        out_type=jax.ShapeDtypeStruct((_NPAD, _NF), jnp.float32),
        scratch_types=[
            pltpu.VMEM((_NCHUNK, _FE, _C), jnp.int32),
            pltpu.VMEM((_FE * _C, _NF), jnp.float32),
            pltpu.VMEM((_C, _NF), jnp.float32),
            pltpu.SemaphoreType.DMA,
        ],
    )
    def sc_fn(p_hbm, idx_hbm, out_hbm, idx_all, rows, outb, sem):
        wid = lax.axis_index("s") * _NC + lax.axis_index("c")
        pltpu.sync_copy(idx_hbm.at[wid], idx_all)

        def chunk(c, carry):
            for k in range(_FE):
                pltpu.async_copy(p_hbm.at[idx_all.at[c, k]],
                                 rows.at[pl.ds(k * _C, _C)], sem)
            pltpu.make_async_copy(p_hbm.at[pl.ds(0, _FE * _C)], rows, sem).wait()

            def vert(i, acc_carry):
                for j in range(_NF // 16):
                    sl = pl.ds(j * 16, 16)
                    acc = rows[i, sl]
                    for k in range(1, _FE):
                        acc = acc + rows[k * _C + i, sl]
                    outb[i, sl] = acc
                return acc_carry

            lax.fori_loop(0, _C, vert, 0)
            pltpu.sync_copy(outb, out_hbm.at[pl.ds(wid * _BPW + c * _C, _C)])
            return carry

        lax.fori_loop(0, _NCHUNK, chunk, 0)

    return sc_fn(p_flat, idx_l)


def kernel(lattice_coarse_values, neighbor_indices, weight):
    table = lattice_coarse_values
    w2 = weight.reshape(_FE, _VAL_DIM, _NF).transpose(1, 0, 2).reshape(
        _VAL_DIM, _FE * _NF)
    p2 = _project_table(table, w2)
    p_flat = p2.reshape(_N_COARSE * _FE, _NF)

    idx = neighbor_indices.astype(jnp.int32)
    idx2 = idx * _FE + jnp.arange(_FE, dtype=jnp.int32)[None, :]
    idx2 = jnp.concatenate(
        [idx2, jnp.zeros((_NPAD - _N_FINE, _FE), jnp.int32)], axis=0)
    idx_l = idx2.reshape(_NW, _NCHUNK, _C, _FE).transpose(0, 1, 3, 2)

    out = _sc_gather_sum(p_flat, idx_l)
    return out[:_N_FINE]


# speedup vs baseline: 6.2194x; 6.2194x over previous
import functools

import jax
import jax.numpy as jnp
from jax import lax
from jax.experimental import pallas as pl
from jax.experimental.pallas import tpu as pltpu
from jax.experimental.pallas import tpu_sc as plsc

_N_COARSE = 10000
_N_FINE = 50000
_VAL_DIM = 128
_FE = 9
_NF = 64

_NC = 2
_NS = 16
_NW = _NC * _NS
_BPW = 1568
_NPAD = _NW * _BPW
_C = 56
_NCHUNK = _BPW // _C


def _mm_body(x_ref, w_ref, o_ref):
    o_ref[...] = jnp.dot(x_ref[...], w_ref[...],
                         preferred_element_type=jnp.float32)


def _project_table(table, w2):
    return pl.pallas_call(
        _mm_body,
        grid=(10,),
        in_specs=[
            pl.BlockSpec((1000, _VAL_DIM), lambda i: (i, 0)),
            pl.BlockSpec((_VAL_DIM, _FE * _NF), lambda i: (0, 0)),
        ],
        out_specs=pl.BlockSpec((1000, _FE * _NF), lambda i: (i, 0)),
        out_shape=jax.ShapeDtypeStruct((_N_COARSE, _FE * _NF), jnp.float32),
    )(table, w2)


def _sc_gather_sum(p_flat, idx_l):
    mesh = plsc.VectorSubcoreMesh(core_axis_name="c", subcore_axis_name="s")

    @functools.partial(
        pl.kernel,
        mesh=mesh,
        out_type=jax.ShapeDtypeStruct((_NPAD, _NF), jnp.float32),
        scratch_types=[
            pltpu.VMEM((_NCHUNK, _FE, _C), jnp.int32),
            pltpu.VMEM((_FE * _C, _NF), jnp.float32),
            pltpu.VMEM((_C, _NF), jnp.float32),
            pltpu.SemaphoreType.DMA,
        ],
        compiler_params=pltpu.CompilerParams(use_tc_tiling_on_sc=False),
    )
    def sc_fn(p_hbm, idx_hbm, out_hbm, idx_all, rows, outb, sem):
        wid = lax.axis_index("s") * _NC + lax.axis_index("c")
        pltpu.sync_copy(idx_hbm.at[wid], idx_all)

        def chunk(c, carry):
            for k in range(_FE):
                pltpu.async_copy(p_hbm.at[idx_all.at[c, k]],
                                 rows.at[pl.ds(k * _C, _C)], sem)
            pltpu.make_async_copy(p_hbm.at[pl.ds(0, _FE * _C)], rows, sem).wait()

            def vert(i, acc_carry):
                for j in range(_NF // 16):
                    sl = pl.ds(j * 16, 16)
                    acc = rows[i, sl]
                    for k in range(1, _FE):
                        acc = acc + rows[k * _C + i, sl]
                    outb[i, sl] = acc
                return acc_carry

            lax.fori_loop(0, _C, vert, 0)
            pltpu.sync_copy(outb, out_hbm.at[pl.ds(wid * _BPW + c * _C, _C)])
            return carry

        lax.fori_loop(0, _NCHUNK, chunk, 0)

    return sc_fn(p_flat, idx_l)


def kernel(lattice_coarse_values, neighbor_indices, weight):
    table = lattice_coarse_values
    w2 = weight.reshape(_FE, _VAL_DIM, _NF).transpose(1, 0, 2).reshape(
        _VAL_DIM, _FE * _NF)
    p2 = _project_table(table, w2)
    p_flat = p2.reshape(_N_COARSE * _FE, _NF)

    idx = neighbor_indices.astype(jnp.int32)
    idx2 = idx * _FE + jnp.arange(_FE, dtype=jnp.int32)[None, :]
    idx2 = jnp.concatenate(
        [idx2, jnp.zeros((_NPAD - _N_FINE, _FE), jnp.int32)], axis=0)
    idx_l = idx2.reshape(_NW, _NCHUNK, _C, _FE).transpose(0, 1, 3, 2)

    out = _sc_gather_sum(p_flat, idx_l)
    return out[:_N_FINE]


# speedup vs baseline: 7.3068x; 1.1748x over previous
import functools

import jax
import jax.numpy as jnp
from jax import lax
from jax.experimental import pallas as pl
from jax.experimental.pallas import tpu as pltpu
from jax.experimental.pallas import tpu_sc as plsc

_N_COARSE = 10000
_N_FINE = 50000
_VAL_DIM = 128
_FE = 9
_NF = 64

_NC = 2
_NS = 16
_NW = _NC * _NS
_BPW = 1568
_NPAD = _NW * _BPW
_C = 56
_NCHUNK = _BPW // _C


def _mm_body(x_ref, w_ref, o_ref):
    o_ref[...] = jnp.dot(x_ref[...], w_ref[...],
                         preferred_element_type=jnp.float32)


def _project_table(table, w2):
    return pl.pallas_call(
        _mm_body,
        grid=(10,),
        in_specs=[
            pl.BlockSpec((1000, _VAL_DIM), lambda i: (i, 0)),
            pl.BlockSpec((_VAL_DIM, _FE * _NF), lambda i: (0, 0)),
        ],
        out_specs=pl.BlockSpec((1000, _FE * _NF), lambda i: (i, 0)),
        out_shape=jax.ShapeDtypeStruct((_N_COARSE, _FE * _NF), jnp.float32),
    )(table, w2)


def _sc_gather_sum(p_flat, idx_l):
    mesh = plsc.VectorSubcoreMesh(core_axis_name="c", subcore_axis_name="s")

    @functools.partial(
        pl.kernel,
        mesh=mesh,
        out_type=jax.ShapeDtypeStruct((_NPAD, _NF), jnp.float32),
        scratch_types=[
            pltpu.VMEM((_NCHUNK, _FE, _C), jnp.int32),
            pltpu.VMEM((_FE * _C, _NF), jnp.float32),
            pltpu.VMEM((_FE * _C, _NF), jnp.float32),
            pltpu.VMEM((_C, _NF), jnp.float32),
            pltpu.VMEM((_C, _NF), jnp.float32),
            pltpu.SemaphoreType.DMA,
            pltpu.SemaphoreType.DMA,
            pltpu.SemaphoreType.DMA,
            pltpu.SemaphoreType.DMA,
        ],
        compiler_params=pltpu.CompilerParams(use_tc_tiling_on_sc=False),
    )
    def sc_fn(p_hbm, idx_hbm, out_hbm, idx_all,
              rows0, rows1, outb0, outb1, sg0, sg1, so0, so1):
        wid = lax.axis_index("s") * _NC + lax.axis_index("c")
        pltpu.sync_copy(idx_hbm.at[wid], idx_all)

        def fire(c, rows, sg):
            for k in range(_FE):
                pltpu.async_copy(p_hbm.at[idx_all.at[c, k]],
                                 rows.at[pl.ds(k * _C, _C)], sg)

        def wait_rows(rows, sg):
            pltpu.make_async_copy(p_hbm.at[pl.ds(0, _FE * _C)], rows, sg).wait()

        def wait_store(outb, so):
            pltpu.make_async_copy(outb, out_hbm.at[pl.ds(0, _C)], so).wait()

        def accum(rows, outb):
            @plsc.parallel_loop(0, _C)
            def _(i):
                for j in range(_NF // 16):
                    sl = pl.ds(j * 16, 16)
                    acc = rows[i, sl]
                    for k in range(1, _FE):
                        acc = acc + rows[k * _C + i, sl]
                    outb[i, sl] = acc

        bufs = ((rows0, outb0, sg0, so0), (rows1, outb1, sg1, so1))
        fire(0, rows0, sg0)
        fire(1, rows1, sg1)

        def step(t, carry):
            for b, (rows, outb, sg, so) in enumerate(bufs):
                c = 2 * t + b
                wait_rows(rows, sg)

                @pl.when(t > 0)
                def _():
                    wait_store(outb, so)

                accum(rows, outb)
                pltpu.async_copy(
                    outb, out_hbm.at[pl.ds(wid * _BPW + c * _C, _C)], so)

                @pl.when(t < _NCHUNK // 2 - 1)
                def _():
                    fire(c + 2, rows, sg)
            return carry

        lax.fori_loop(0, _NCHUNK // 2, step, 0)
        wait_store(outb0, so0)
        wait_store(outb1, so1)

    return sc_fn(p_flat, idx_l)


def kernel(lattice_coarse_values, neighbor_indices, weight):
    table = lattice_coarse_values
    w2 = weight.reshape(_FE, _VAL_DIM, _NF).transpose(1, 0, 2).reshape(
        _VAL_DIM, _FE * _NF)
    p2 = _project_table(table, w2)
    p_flat = p2.reshape(_N_COARSE * _FE, _NF)

    idx = neighbor_indices.astype(jnp.int32)
    idx2 = idx * _FE + jnp.arange(_FE, dtype=jnp.int32)[None, :]
    idx2 = jnp.concatenate(
        [idx2, jnp.zeros((_NPAD - _N_FINE, _FE), jnp.int32)], axis=0)
    idx_l = idx2.reshape(_NW, _NCHUNK, _C, _FE).transpose(0, 1, 3, 2)

    out = _sc_gather_sum(p_flat, idx_l)
    return out[:_N_FINE]


# speedup vs baseline: 10.4691x; 1.4328x over previous
import functools

import jax
import jax.numpy as jnp
from jax import lax
from jax.experimental import pallas as pl
from jax.experimental.pallas import tpu as pltpu
from jax.experimental.pallas import tpu_sc as plsc

_N_COARSE = 10000
_N_FINE = 50000
_VAL_DIM = 128
_FE = 9
_NF = 64
_HALF = _N_COARSE // 2

_NC = 2
_NS = 16
_NW = _NC * _NS
_CH = 40
_NCHK = _N_FINE // _CH
_TPW = 40
_PR = _CH // 2
_E = _CH * _FE


def _mm_body(x_ref, w_ref, o_ref):
    wk = w_ref[0]
    a = jnp.dot(x_ref[:_HALF], wk, preferred_element_type=jnp.float32)
    b = jnp.dot(x_ref[_HALF:], wk, preferred_element_type=jnp.float32)
    o_ref[...] = jnp.concatenate([a, b], axis=1)


def _project_table(table, w3):
    return pl.pallas_call(
        _mm_body,
        grid=(_FE,),
        in_specs=[
            pl.BlockSpec((_N_COARSE, _VAL_DIM), lambda k: (0, 0)),
            pl.BlockSpec((1, _VAL_DIM, _NF), lambda k: (k, 0, 0)),
        ],
        out_specs=pl.BlockSpec((_HALF, 2 * _NF), lambda k: (k, 0)),
        out_shape=jax.ShapeDtypeStruct((_FE * _HALF, 2 * _NF), jnp.float32),
    )(table, w3)


def _sc_gather_sum(p_flat, idx_r):
    mesh = plsc.VectorSubcoreMesh(core_axis_name="c", subcore_axis_name="s")

    @functools.partial(
        pl.kernel,
        mesh=mesh,
        out_type=jax.ShapeDtypeStruct((_N_FINE // 2, 2 * _NF), jnp.float32),
        scratch_types=[
            pltpu.VMEM((_TPW, _E), jnp.int32),
            pltpu.VMEM((_E, _NF), jnp.float32),
            pltpu.VMEM((_E, _NF), jnp.float32),
            pltpu.VMEM((_PR, 2 * _NF), jnp.float32),
            pltpu.VMEM((_PR, 2 * _NF), jnp.float32),
            pltpu.SemaphoreType.DMA,
            pltpu.SemaphoreType.DMA,
            pltpu.SemaphoreType.DMA,
            pltpu.SemaphoreType.DMA,
            pltpu.SemaphoreType.DMA,
        ],
        compiler_params=pltpu.CompilerParams(use_tc_tiling_on_sc=False),
    )
    def sc_fn(p_hbm, idx_hbm, out_hbm, idxb,
              rows0, rows1, outb0, outb1, si, sg0, sg1, so0, so1):
        w = lax.axis_index("s") * _NC + lax.axis_index("c")

        def cid_of(t):
            c0 = w + _NW * t
            return jnp.where(c0 < _NCHK, c0, w)

        def pf(t, carry):
            pltpu.async_copy(idx_hbm.at[cid_of(t)], idxb.at[t], si)
            return carry

        lax.fori_loop(0, _TPW, pf, 0)
        pltpu.make_async_copy(idx_hbm.at[pl.ds(0, _TPW)], idxb, si).wait()

        def fire(t, rows, sg):
            for m in range(3):
                sl = pl.ds(m * 120, 120)
                pltpu.async_copy(p_hbm.at[idxb.at[t, sl]], rows.at[sl], sg)

        def wait_rows(rows, sg):
            pltpu.make_async_copy(p_hbm.at[pl.ds(0, _E)], rows, sg).wait()

        def wait_store(outb, so):
            pltpu.make_async_copy(outb, out_hbm.at[pl.ds(0, _PR)], so).wait()

        def accum(rows, outb):
            @plsc.parallel_loop(0, _PR)
            def _(u):
                base = u * (2 * _FE)
                for h in range(2):
                    for j in range(_NF // 16):
                        sj = pl.ds(j * 16, 16)
                        acc = rows[base + h * _FE, sj]
                        for k in range(1, _FE):
                            acc = acc + rows[base + h * _FE + k, sj]
                        outb[u, pl.ds(h * _NF + j * 16, 16)] = acc

        bufs = ((rows0, outb0, sg0, so0), (rows1, outb1, sg1, so1))
        fire(0, rows0, sg0)
        fire(1, rows1, sg1)

        def step(t2, carry):
            for b, (rows, outb, sg, so) in enumerate(bufs):
                t = 2 * t2 + b
                wait_rows(rows, sg)

                @pl.when(t2 > 0)
                def _():
                    wait_store(outb, so)

                accum(rows, outb)
                pltpu.async_copy(
                    outb, out_hbm.at[pl.ds(cid_of(t) * _PR, _PR)], so)

                @pl.when(t2 < _TPW // 2 - 1)
                def _():
                    fire(t + 2, rows, sg)
            return carry

        lax.fori_loop(0, _TPW // 2, step, 0)
        wait_store(outb0, so0)
        wait_store(outb1, so1)

    return sc_fn(p_flat, idx_r)


def kernel(lattice_coarse_values, neighbor_indices, weight):
    table = lattice_coarse_values
    w3 = weight.reshape(_FE, _VAL_DIM, _NF)
    p2 = _project_table(table, w3)
    p_flat = p2.reshape(_N_COARSE * _FE, _NF)

    v = neighbor_indices.astype(jnp.int32)
    v2 = v * 2
    idx2 = jnp.where(v < _HALF, v2, v2 - (_N_COARSE - 1)) + \
        (jnp.arange(_FE, dtype=jnp.int32) * _N_COARSE)[None, :]
    idx_r = idx2.reshape(_NCHK, _E)

    out = _sc_gather_sum(p_flat, idx_r)
    return out.reshape(_N_FINE, _NF)


# speedup vs baseline: 13.3762x; 1.2777x over previous
import functools

import jax
import jax.numpy as jnp
from jax import lax
from jax.experimental import pallas as pl
from jax.experimental.pallas import tpu as pltpu
from jax.experimental.pallas import tpu_sc as plsc

_N_COARSE = 10000
_N_FINE = 50000
_VAL_DIM = 128
_FE = 9
_NF = 64
_HALF = _N_COARSE // 2

_NC = 2
_NS = 16
_NW = _NC * _NS
_BPW = 1568
_C = 56
_NCHUNK = _BPW // _C
_PR = _C // 2


def _mm_body(x_ref, w_ref, o_ref):
    wk = w_ref[0]
    a = jnp.dot(x_ref[:_HALF], wk, preferred_element_type=jnp.float32)
    b = jnp.dot(x_ref[_HALF:], wk, preferred_element_type=jnp.float32)
    o_ref[...] = jnp.concatenate([a, b], axis=1)


def _project_table(table, w3):
    return pl.pallas_call(
        _mm_body,
        grid=(_FE,),
        in_specs=[
            pl.BlockSpec((_N_COARSE, _VAL_DIM), lambda k: (0, 0)),
            pl.BlockSpec((1, _VAL_DIM, _NF), lambda k: (k, 0, 0)),
        ],
        out_specs=pl.BlockSpec((_HALF, 2 * _NF), lambda k: (k, 0)),
        out_shape=jax.ShapeDtypeStruct((_FE * _HALF, 2 * _NF), jnp.float32),
    )(table, w3)


def _sc_gather_sum(p_flat, idx_r):
    mesh = plsc.VectorSubcoreMesh(core_axis_name="c", subcore_axis_name="s")

    @functools.partial(
        pl.kernel,
        mesh=mesh,
        out_type=jax.ShapeDtypeStruct((_N_FINE // 2, 2 * _NF), jnp.float32),
        scratch_types=[
            pltpu.VMEM((_FE, _BPW), jnp.int32),
            pltpu.VMEM((_FE * _C, _NF), jnp.float32),
            pltpu.VMEM((_FE * _C, _NF), jnp.float32),
            pltpu.VMEM((_PR, 2 * _NF), jnp.float32),
            pltpu.VMEM((_PR, 2 * _NF), jnp.float32),
            pltpu.SemaphoreType.DMA,
            pltpu.SemaphoreType.DMA,
            pltpu.SemaphoreType.DMA,
            pltpu.SemaphoreType.DMA,
            pltpu.SemaphoreType.DMA,
        ],
        compiler_params=pltpu.CompilerParams(use_tc_tiling_on_sc=False),
    )
    def sc_fn(p_hbm, idx_hbm, out_hbm, idxb,
              rows0, rows1, outb0, outb1, si, sg0, sg1, so0, so1):
        w = lax.axis_index("s") * _NC + lax.axis_index("c")
        start = jnp.where(w < _NW - 1, w * _BPW, _N_FINE - _BPW)

        for k in range(_FE):
            pltpu.async_copy(idx_hbm.at[pl.ds(k * _N_FINE + start, _BPW)],
                             idxb.at[k], si)
        pltpu.make_async_copy(idx_hbm.at[pl.ds(0, _FE * _BPW)], idxb, si).wait()

        def fire(c, rows, sg):
            for k in range(_FE):
                pltpu.async_copy(p_hbm.at[idxb.at[k, pl.ds(c * _C, _C)]],
                                 rows.at[pl.ds(k * _C, _C)], sg)

        def wait_rows(rows, sg):
            pltpu.make_async_copy(p_hbm.at[pl.ds(0, _FE * _C)], rows, sg).wait()

        def wait_store(outb, so):
            pltpu.make_async_copy(outb, out_hbm.at[pl.ds(0, _PR)], so).wait()

        def accum(rows, outb):
            @plsc.parallel_loop(0, _PR)
            def _(u):
                for h in range(2):
                    i = 2 * u + h
                    for j in range(_NF // 16):
                        sj = pl.ds(j * 16, 16)
                        acc = rows[i, sj]
                        for k in range(1, _FE):
                            acc = acc + rows[k * _C + i, sj]
                        outb[u, pl.ds(h * _NF + j * 16, 16)] = acc

        bufs = ((rows0, outb0, sg0, so0), (rows1, outb1, sg1, so1))
        fire(0, rows0, sg0)
        fire(1, rows1, sg1)

        def step(t2, carry):
            for b, (rows, outb, sg, so) in enumerate(bufs):
                c = 2 * t2 + b
                wait_rows(rows, sg)

                @pl.when(t2 > 0)
                def _():
                    wait_store(outb, so)

                accum(rows, outb)
                pltpu.async_copy(
                    outb,
                    out_hbm.at[pl.ds(start // 2 + c * _PR, _PR)], so)

                @pl.when(t2 < _NCHUNK // 2 - 1)
                def _():
                    fire(c + 2, rows, sg)
            return carry

        lax.fori_loop(0, _NCHUNK // 2, step, 0)
        wait_store(outb0, so0)
        wait_store(outb1, so1)

    return sc_fn(p_flat, idx_r)


def kernel(lattice_coarse_values, neighbor_indices, weight):
    table = lattice_coarse_values
    w3 = weight.reshape(_FE, _VAL_DIM, _NF)
    p2 = _project_table(table, w3)
    p_flat = p2.reshape(_N_COARSE * _FE, _NF)

    v = neighbor_indices.T.astype(jnp.int32)
    v2 = v * 2
    idx2 = jnp.where(v < _HALF, v2, v2 - (_N_COARSE - 1)) + \
        (jnp.arange(_FE, dtype=jnp.int32) * _N_COARSE)[:, None]
    idx_r = idx2.reshape(_FE * _N_FINE)

    out = _sc_gather_sum(p_flat, idx_r)
    return out.reshape(_N_FINE, _NF)
